# baseline (speedup 1.0000x reference)
import functools

import jax
import jax.numpy as jnp
from jax import lax
from jax.experimental import pallas as pl
from jax.experimental.pallas import tpu as pltpu
from jax.experimental.pallas import tpu_sc as plsc

B = 16384
DIM = 128
NB = 32
ANC = 16
NID = NB + ANC
NTOK = 1 + NID
SCALE = 1.0 / NTOK

LANES = 16
VPR = DIM // LANES

_INFO = plsc.get_sparse_core_info()
NC = _INFO.num_cores
NS = _INFO.num_subcores
NW = NC * NS
BPW = B // NW
C = 8
NCH = BPW // C
NIDS = 4

_mesh = plsc.VectorSubcoreMesh(core_axis_name="c", subcore_axis_name="s")


@functools.partial(
    pl.kernel,
    out_type=jax.ShapeDtypeStruct((B, DIM), jnp.float32),
    mesh=_mesh,
    scratch_types=[
        pltpu.VMEM((BPW + LANES,), jnp.int32),
        pltpu.VMEM((NIDS, C, NID), jnp.int32),
        pltpu.VMEM((2, C, DIM), jnp.float32),
        pltpu.VMEM((2, C, NID, DIM), jnp.float32),
        pltpu.VMEM((2, C, DIM), jnp.float32),
        pltpu.SemaphoreType.DMA,
        pltpu.SemaphoreType.DMA,
        pltpu.SemaphoreType.DMA,
        pltpu.SemaphoreType.DMA,
        pltpu.SemaphoreType.DMA,
        pltpu.SemaphoreType.DMA,
        pltpu.SemaphoreType.DMA,
    ],
)
def _sc_combine(ents_hbm, nodes_hbm, hashes_hbm, emb_hbm, out_hbm,
                ent_v, ids_v, ctr_v, tok_v, out_v,
                sem_id0, sem_id1, sem_id2, sem_id3,
                sem_emb0, sem_emb1, sem_out):
    sem_id = (sem_id0, sem_id1, sem_id2, sem_id3)
    sem_emb = (sem_emb0, sem_emb1)
    wid = lax.axis_index("s") * NC + lax.axis_index("c")
    base = wid * BPW

    pltpu.sync_copy(ents_hbm.at[pl.ds(base, BPW)],
                    ent_v.at[pl.ds(0, BPW)])

    def id_copies(t, ev):
        cps = []
        for e in range(C):
            ent = 0 if ev is None else ev[e]
            cps.append(pltpu.make_async_copy(
                nodes_hbm.at[ent], ids_v.at[t, e, pl.ds(0, NB)], sem_id[t]))
            cps.append(pltpu.make_async_copy(
                hashes_hbm.at[ent], ids_v.at[t, e, pl.ds(NB, ANC)],
                sem_id[t]))
        return cps

    def issue_ids(ci, t):
        ev = ent_v[pl.ds(ci * C, LANES)]
        for cp in id_copies(t, ev):
            cp.start()

    def wait_ids(t):
        for cp in id_copies(t, None):
            cp.wait()

    def emb_copies(ci, t, b):
        cps = [pltpu.make_async_copy(
            emb_hbm.at[ent_v.at[pl.ds(ci * C, C)]], ctr_v.at[b], sem_emb[b])]
        for e in range(C):
            cps.append(pltpu.make_async_copy(
                emb_hbm.at[ids_v.at[t, e]], tok_v.at[b, e], sem_emb[b]))
        return cps

    def issue_emb(ci, t, b):
        for cp in emb_copies(ci, t, b):
            cp.start()

    def wait_emb(b):
        for cp in emb_copies(0, 0, b):
            cp.wait()

    def out_copy(ci, b):
        return pltpu.make_async_copy(
            out_v.at[b], out_hbm.at[pl.ds(base + ci * C, C)], sem_out)

    def compute(ci, b):
        for e in range(C):
            accs = tuple(ctr_v[b, e, pl.ds(k * LANES, LANES)]
                         for k in range(VPR))

            def step(j, a):
                return tuple(a[k] + tok_v[b, e, j, pl.ds(k * LANES, LANES)]
                             for k in range(VPR))

            accs = lax.fori_loop(0, NID, step, accs)
            for k in range(VPR):
                out_v[b, e, pl.ds(k * LANES, LANES)] = accs[k] * SCALE
        out_copy(ci, b).start()

    issue_ids(0, 0)
    issue_ids(1, 1)
    issue_ids(2, 2)
    wait_ids(0)
    issue_emb(0, 0, 0)

    def outer(g, carry):
        for u in range(NIDS):
            ci = NIDS * g + u
            b = u % 2

            @pl.when(ci + 3 < NCH)
            def _():
                issue_ids(ci + 3, (u + 3) % NIDS)

            @pl.when(ci + 1 < NCH)
            def _():
                wait_ids((u + 1) % NIDS)
                issue_emb(ci + 1, (u + 1) % NIDS, 1 - b)

            wait_emb(b)

            @pl.when(ci >= 2)
            def _():
                out_copy(ci - 2, b).wait()

            compute(ci, b)
        return carry

    lax.fori_loop(0, NCH // NIDS, outer, 0)
    out_copy(NCH - 2, 0).wait()
    out_copy(NCH - 1, 1).wait()


def kernel(entities, nodes, hashes, node_embedding):
    return _sc_combine(entities, nodes, hashes, node_embedding)


# speedup vs baseline: 1.0114x; 1.0114x over previous
import functools

import jax
import jax.numpy as jnp
from jax import lax
from jax.experimental import pallas as pl
from jax.experimental.pallas import tpu as pltpu
from jax.experimental.pallas import tpu_sc as plsc

B = 16384
DIM = 128
NB = 32
ANC = 16
NID = NB + ANC
NTOK = 1 + NID
SCALE = 1.0 / NTOK

LANES = 16
VPR = DIM // LANES

_INFO = plsc.get_sparse_core_info()
NC = _INFO.num_cores
NS = _INFO.num_subcores
NW = NC * NS
BPW = B // NW
C = 8
NCH = BPW // C

_mesh = plsc.VectorSubcoreMesh(core_axis_name="c", subcore_axis_name="s")


@functools.partial(
    pl.kernel,
    out_type=jax.ShapeDtypeStruct((B, DIM), jnp.float32),
    mesh=_mesh,
    scratch_types=[
        pltpu.VMEM((BPW + LANES,), jnp.int32),
        pltpu.VMEM((2, C, NID), jnp.int32),
        pltpu.VMEM((2, C, DIM), jnp.float32),
        pltpu.VMEM((2, C, NID, DIM), jnp.float32),
        pltpu.VMEM((2, C, DIM), jnp.float32),
        pltpu.SemaphoreType.DMA,
        pltpu.SemaphoreType.DMA,
        pltpu.SemaphoreType.DMA,
        pltpu.SemaphoreType.DMA,
        pltpu.SemaphoreType.DMA,
    ],
)
def _sc_combine(ents_hbm, nodes_hbm, hashes_hbm, emb_hbm, out_hbm,
                ent_v, ids_v, ctr_v, tok_v, out_v,
                sem_id0, sem_id1, sem_emb0, sem_emb1, sem_out):
    sem_id = (sem_id0, sem_id1)
    sem_emb = (sem_emb0, sem_emb1)
    wid = lax.axis_index("s") * NC + lax.axis_index("c")
    base = wid * BPW

    pltpu.sync_copy(ents_hbm.at[pl.ds(base, BPW)],
                    ent_v.at[pl.ds(0, BPW)])

    def id_copies(b, ev):
        cps = []
        for e in range(C):
            ent = 0 if ev is None else ev[e]
            cps.append(pltpu.make_async_copy(
                nodes_hbm.at[ent], ids_v.at[b, e, pl.ds(0, NB)], sem_id[b]))
            cps.append(pltpu.make_async_copy(
                hashes_hbm.at[ent], ids_v.at[b, e, pl.ds(NB, ANC)],
                sem_id[b]))
        return cps

    def issue_ids(ci, b):
        ev = ent_v[pl.ds(ci * C, LANES)]
        for cp in id_copies(b, ev):
            cp.start()

    def wait_ids(b):
        for cp in id_copies(b, None):
            cp.wait()

    def emb_copies(ci, b):
        cps = [pltpu.make_async_copy(
            emb_hbm.at[ent_v.at[pl.ds(ci * C, C)]], ctr_v.at[b], sem_emb[b])]
        for e in range(C):
            cps.append(pltpu.make_async_copy(
                emb_hbm.at[ids_v.at[b, e]], tok_v.at[b, e], sem_emb[b]))
        return cps

    def issue_emb(ci, b):
        for cp in emb_copies(ci, b):
            cp.start()

    def wait_emb(b):
        for cp in emb_copies(0, b):
            cp.wait()

    def out_copy(ci, b):
        return pltpu.make_async_copy(
            out_v.at[b], out_hbm.at[pl.ds(base + ci * C, C)], sem_out)

    def compute(ci, b):
        for e in range(C):
            accs = tuple(ctr_v[b, e, pl.ds(k * LANES, LANES)]
                         for k in range(VPR))

            def step(j, a):
                return tuple(a[k] + tok_v[b, e, j, pl.ds(k * LANES, LANES)]
                             for k in range(VPR))

            accs = lax.fori_loop(0, NID, step, accs)
            for k in range(VPR):
                out_v[b, e, pl.ds(k * LANES, LANES)] = accs[k] * SCALE
        out_copy(ci, b).start()

    issue_ids(0, 0)
    wait_ids(0)
    issue_emb(0, 0)
    issue_ids(1, 1)

    def outer(g, carry):
        for b in range(2):
            ci = 2 * g + b

            @pl.when(ci + 1 < NCH)
            def _():
                wait_ids(1 - b)
                issue_emb(ci + 1, 1 - b)

            wait_emb(b)

            @pl.when(ci + 2 < NCH)
            def _():
                issue_ids(ci + 2, b)

            @pl.when(ci >= 2)
            def _():
                out_copy(ci - 2, b).wait()

            compute(ci, b)
        return carry

    lax.fori_loop(0, NCH // 2, outer, 0)
    out_copy(NCH - 2, 0).wait()
    out_copy(NCH - 1, 1).wait()


def kernel(entities, nodes, hashes, node_embedding):
    return _sc_combine(entities, nodes, hashes, node_embedding)
